# baseline (speedup 1.0000x reference)
import functools

import jax
import jax.numpy as jnp
from jax import lax
from jax.experimental import pallas as pl
from jax.experimental.pallas import tpu as pltpu
from jax.experimental.pallas import tpu_sc as plsc

DEPTH = 1000
N_ROWS = 4096 * 26
FLAT = N_ROWS * DEPTH
FILL_COLS = 1024
FILL_ROWS = FLAT // FILL_COLS
FILL_BLOCK_ROWS = 2000
N_TILES = 32
PER_TILE = N_ROWS // N_TILES
CHUNK = 128
N_CHUNKS = PER_TILE // CHUNK
LANES = 16


def _zero_block(o_ref):
    o_ref[...] = jnp.zeros_like(o_ref)


def _tc_zero_fill():
    return pl.pallas_call(
        _zero_block,
        grid=(FILL_ROWS // FILL_BLOCK_ROWS,),
        out_specs=pl.BlockSpec((FILL_BLOCK_ROWS, FILL_COLS), lambda i: (i, 0)),
        out_shape=jax.ShapeDtypeStruct((FILL_ROWS, FILL_COLS), jnp.float32),
    )()


_MESH = plsc.VectorSubcoreMesh(core_axis_name="c", subcore_axis_name="s")


@functools.partial(
    pl.kernel,
    out_type=(),
    mesh=_MESH,
    scratch_types=[
        pltpu.VMEM((PER_TILE,), jnp.int32),
        pltpu.VMEM((N_CHUNKS, CHUNK), jnp.int32),
        pltpu.VMEM((CHUNK,), jnp.float32),
        pltpu.SemaphoreType.DMA,
    ],
)
def _sc_scatter(idx_hbm, buf_hbm, idx_v, pos_v, ones_v, sem):
    wid = lax.axis_index("s") * 2 + lax.axis_index("c")
    base = wid * PER_TILE
    pltpu.sync_copy(idx_hbm.at[pl.ds(base, PER_TILE)], idx_v)
    for k in range(CHUNK // LANES):
        ones_v[pl.ds(k * LANES, LANES)] = jnp.full((LANES,), 1.0, jnp.float32)
    for j in range(N_CHUNKS):
        def body(l, _, j=j):
            off = j * CHUNK + l * LANES
            pos = (idx_v[pl.ds(off, LANES)]
                   + (base + off) * DEPTH
                   + lax.iota(jnp.int32, LANES) * DEPTH)
            pos_v[j, pl.ds(l * LANES, LANES)] = pos
            return 0
        lax.fori_loop(0, CHUNK // LANES, body, 0)
    descs = [
        pltpu.async_copy(ones_v, buf_hbm.at[pos_v.at[j]], sem)
        for j in range(N_CHUNKS)
    ]
    for d in descs:
        d.wait()


def kernel(inputs):
    flat_idx = inputs.reshape(FLAT // DEPTH)
    buf = jax.new_ref(_tc_zero_fill().reshape(FLAT))
    _sc_scatter(flat_idx, buf)
    return jax.freeze(buf).reshape(4096, 26, DEPTH)
